# baseline (speedup 1.0000x reference)
import functools

import jax
import jax.numpy as jnp
from jax import lax
from jax.experimental import pallas as pl
from jax.experimental.pallas import tpu as pltpu
from jax.experimental.pallas import tpu_sc as plsc

N = 10000
E = 160000
IN = 128
HG = 256
HT = 256
B = 2
T = 4
P = B * T
RB = 400

NC, NS = 2, 16
NPAD = 10240
RPS = NPAD // NS
CH = 128
EP = 163840
EH = EP // 2
NCH = 40
KB = 4
PF = 2
ZR = 64
FW = 128
TBL = NC * NPAD
HOFF = P * N + (TBL - N)
NPH = NPAD // 2
NPH_PAD = NPH + CH
RPSH = NPH_PAD // NS
FPSH = NPH // NS

_MESH = plsc.VectorSubcoreMesh(core_axis_name="c", subcore_axis_name="s")
_SC_PARAMS = pltpu.CompilerParams(use_tc_tiling_on_sc=False)


def _agg_body(tab_hbm, idx_hbm, dst_hbm, out_hbm, gidx, didx0, didx1, rows,
              zbuf, acc, *sems):
    gsem, ssem = sems[:KB], sems[KB:]
    c = lax.axis_index("c")
    s = lax.axis_index("s")

    @pl.loop(0, ZR)
    def _zrow(r):
        @pl.loop(0, FW // 16)
        def _zcol(k):
            zbuf[r, pl.ds(k * 16, 16)] = jnp.zeros((16,), jnp.float32)

    pltpu.sync_copy(idx_hbm.at[c, s], gidx)
    pltpu.sync_copy(dst_hbm.at[0, c, s], didx0)
    pltpu.sync_copy(dst_hbm.at[1, c, s], didx1)

    for q in range(2):
        for off in range(0, RPSH, ZR):
            sz = min(ZR, RPSH - off)
            pltpu.sync_copy(zbuf.at[pl.ds(0, sz)],
                            acc.at[pl.ds(s * RPSH + off, sz)])

        plsc.subcore_barrier()
        dq = didx0 if q == 0 else didx1

        for m in range(PF):
            pltpu.async_copy(tab_hbm.at[gidx.at[m]], rows.at[m], gsem[m])

        @pl.loop(0, NCH // KB)
        def _grp(jj):
            for b in range(KB):
                j = jj * KB + b
                pltpu.make_async_copy(tab_hbm.at[gidx.at[j]], rows.at[b],
                                      gsem[b]).wait()
                pltpu.async_copy(rows.at[b], acc.at[dq.at[j]], ssem[b],
                                 add=True)
                bm = (b + PF) % KB

                @pl.when(j + PF < NCH)
                def _next():
                    @pl.when(j + PF >= KB)
                    def _wait_prev():
                        pltpu.make_async_copy(rows.at[bm], acc.at[dq.at[0]],
                                              ssem[bm]).wait()

                    pltpu.async_copy(tab_hbm.at[gidx.at[j + PF]],
                                     rows.at[bm], gsem[bm])

        for b in range(KB):
            pltpu.make_async_copy(rows.at[b], acc.at[dq.at[0]],
                                  ssem[b]).wait()

        plsc.subcore_barrier()
        pltpu.sync_copy(acc.at[pl.ds(s * FPSH, FPSH)],
                        out_hbm.at[c, pl.ds(q * NPH + s * FPSH, FPSH)])
        plsc.subcore_barrier()


_agg_step = pl.kernel(
    _agg_body,
    out_type=jax.ShapeDtypeStruct((NC, NPAD, FW), jnp.float32),
    mesh=_MESH,
    scratch_types=[
        pltpu.VMEM((NCH, CH), jnp.int32),
        pltpu.VMEM((NCH, CH), jnp.int32),
        pltpu.VMEM((NCH, CH), jnp.int32),
        pltpu.VMEM((KB, CH, FW), jnp.float32),
        pltpu.VMEM((ZR, FW), jnp.float32),
        pltpu.VMEM_SHARED((NPH_PAD, FW), jnp.float32),
    ] + [pltpu.SemaphoreType.DMA] * (2 * KB),
    compiler_params=_SC_PARAMS,
)


def _dense1_body(s_ref, deg_ref, x_ref, wl_ref, wr_ref, b_ref, g_ref, be_ref,
                 o_ref):
    deg = jnp.maximum(deg_ref[...], 1.0)
    mean = (s_ref[0] + s_ref[1]) / deg
    z = (jnp.dot(mean, wl_ref[...], preferred_element_type=jnp.float32)
         + jnp.dot(x_ref[...], wr_ref[...], preferred_element_type=jnp.float32)
         + b_ref[...])
    mu = jnp.mean(z, axis=-1, keepdims=True)
    var = jnp.mean((z - mu) ** 2, axis=-1, keepdims=True)
    z = (z - mu) * jax.lax.rsqrt(var + 1e-5) * g_ref[...] + be_ref[...]
    z = jnp.maximum(z, 0.0)
    o_ref[0] = z[:, :128]
    o_ref[1] = z[:, 128:]


def _dense1(s, deg, x, wl, wr, b, g, be):
    return pl.pallas_call(
        _dense1_body,
        grid=(N // RB,),
        in_specs=[
            pl.BlockSpec((NC, RB, 128), lambda r: (0, r, 0)),
            pl.BlockSpec((RB, 1), lambda r: (r, 0)),
            pl.BlockSpec((RB, IN), lambda r: (r, 0)),
            pl.BlockSpec((IN, HG), lambda r: (0, 0)),
            pl.BlockSpec((IN, HG), lambda r: (0, 0)),
            pl.BlockSpec((1, HG), lambda r: (0, 0)),
            pl.BlockSpec((1, HG), lambda r: (0, 0)),
            pl.BlockSpec((1, HG), lambda r: (0, 0)),
        ],
        out_specs=pl.BlockSpec((NC, RB, 128), lambda r: (0, r, 0)),
        out_shape=jax.ShapeDtypeStruct((NC, NPAD, 128), jnp.float32),
    )(s, deg, x, wl, wr, b, g, be)


def _dense2_body(sa_ref, sb_ref, deg_ref, h_ref, wl_ref, wr_ref, b_ref,
                 g_ref, be_ref, o_ref):
    r = pl.program_id(0)
    deg = jnp.maximum(deg_ref[...], 1.0)
    mean = (jnp.concatenate([sa_ref[0], sa_ref[1]], axis=1)
            + jnp.concatenate([sb_ref[0], sb_ref[1]], axis=1)) / deg
    h = jnp.concatenate([h_ref[0], h_ref[1]], axis=1)
    z = (jnp.dot(mean, wl_ref[...], preferred_element_type=jnp.float32)
         + jnp.dot(h, wr_ref[...], preferred_element_type=jnp.float32)
         + b_ref[...])
    mu = jnp.mean(z, axis=-1, keepdims=True)
    var = jnp.mean((z - mu) ** 2, axis=-1, keepdims=True)
    z = (z - mu) * jax.lax.rsqrt(var + 1e-5) * g_ref[...] + be_ref[...]
    z = jnp.maximum(z, 0.0)
    part = jnp.sum(z, axis=0, keepdims=True)[None] * (1.0 / N)

    @pl.when(r == 0)
    def _():
        o_ref[...] = jnp.zeros_like(o_ref)

    o_ref[...] += part


def _dense2_pool(sa, sb, deg, h, wl, wr, b, g, be):
    return pl.pallas_call(
        _dense2_body,
        grid=(N // RB,),
        in_specs=[
            pl.BlockSpec((NC, RB, 128), lambda r: (0, r, 0)),
            pl.BlockSpec((NC, RB, 128), lambda r: (0, r, 0)),
            pl.BlockSpec((RB, 1), lambda r: (r, 0)),
            pl.BlockSpec((NC, RB, 128), lambda r: (0, r, 0)),
            pl.BlockSpec((HG, HG), lambda r: (0, 0)),
            pl.BlockSpec((HG, HG), lambda r: (0, 0)),
            pl.BlockSpec((1, HG), lambda r: (0, 0)),
            pl.BlockSpec((1, HG), lambda r: (0, 0)),
            pl.BlockSpec((1, HG), lambda r: (0, 0)),
        ],
        out_specs=pl.BlockSpec((1, 1, HG), lambda r: (0, 0, 0)),
        out_shape=jax.ShapeDtypeStruct((1, 1, HG), jnp.float32),
    )(sa, sb, deg, h, wl, wr, b, g, be)


def _gru_body(h_ref, wih_ref, whh_ref, bih_ref, bhh_ref, wh_ref, bh_ref,
              o_ref):
    h = jnp.zeros((B, HT), jnp.float32)
    for t in range(T):
        xt = h_ref[:, t, :]
        gi = jnp.dot(xt, wih_ref[...],
                     preferred_element_type=jnp.float32) + bih_ref[...]
        gh = jnp.dot(h, whh_ref[...],
                     preferred_element_type=jnp.float32) + bhh_ref[...]
        ir, iz, inn = gi[:, :HT], gi[:, HT:2 * HT], gi[:, 2 * HT:]
        hr, hz, hn = gh[:, :HT], gh[:, HT:2 * HT], gh[:, 2 * HT:]
        r = jax.nn.sigmoid(ir + hr)
        z = jax.nn.sigmoid(iz + hz)
        n = jnp.tanh(inn + r * hn)
        h = (1.0 - z) * n + z * h
    o_ref[...] = (jnp.dot(h, wh_ref[...], preferred_element_type=jnp.float32)
                  + bh_ref[...])


def _gru_head(Hseq, wihT, whhT, bih, bhh, whp, bhp):
    return pl.pallas_call(
        _gru_body,
        in_specs=[pl.BlockSpec(Hseq.shape, lambda: (0, 0, 0)),
                  pl.BlockSpec(wihT.shape, lambda: (0, 0)),
                  pl.BlockSpec(whhT.shape, lambda: (0, 0)),
                  pl.BlockSpec(bih.shape, lambda: (0, 0)),
                  pl.BlockSpec(bhh.shape, lambda: (0, 0)),
                  pl.BlockSpec(whp.shape, lambda: (0, 0)),
                  pl.BlockSpec(bhp.shape, lambda: (0, 0))],
        out_specs=pl.BlockSpec((B, 128), lambda: (0, 0)),
        out_shape=jax.ShapeDtypeStruct((B, 128), jnp.float32),
    )(Hseq, wihT, whhT, bih, bhh, whp, bhp)


def kernel(x_seq, edge_index, Wl1, bl1, Wr1, br1, g1, be1, Wl2, bl2, Wr2, br2,
           g2, be2, W_ih, W_hh, b_ih, b_hh, Wh, bh):
    src = edge_index[0].astype(jnp.int32)
    dst = edge_index[1].astype(jnp.int32)
    srcp = jnp.pad(src, (0, EP - E))
    dstp = jnp.pad(dst, (0, EP - E), constant_values=N)

    cc = jnp.arange(NC, dtype=jnp.int32)[:, None]
    ee = jnp.arange(EP, dtype=jnp.int32)
    idx_l1 = srcp.reshape(NC, NS, NCH, CH)
    idx_2a = (cc * NPAD + srcp[None, :EH]).reshape(NC, NS, NCH, CH)
    idx_2b = (cc * NPAD + srcp[None, EH:]).reshape(NC, NS, NCH, CH)
    idx_dg = (ee % CH).reshape(NC, NS, NCH, CH)
    IDX = jnp.stack([idx_l1, idx_2a, idx_2b, idx_dg])

    qq = jnp.arange(2, dtype=jnp.int32)[:, None]
    owned = (dstp[None, :] >= qq * NPH) & (dstp[None, :] < (qq + 1) * NPH)
    dstq = jnp.where(owned, dstp[None, :] - qq * NPH,
                     NPH + (ee[None, :] % CH))
    dst_l1 = dstq.reshape(2, NC, NS, NCH, CH)
    dst_2a = jnp.broadcast_to(dstq[:, :EH].reshape(2, 1, NS, NCH, CH),
                              (2, NC, NS, NCH, CH))
    dst_2b = jnp.broadcast_to(dstq[:, EH:].reshape(2, 1, NS, NCH, CH),
                              (2, NC, NS, NCH, CH))
    DST = jnp.stack([dst_l1, dst_2a, dst_2b, dst_l1])

    b1 = (bl1 + br1)[None]
    b2 = (bl2 + br2)[None]
    g1b, be1b, g2b, be2b = g1[None], be1[None], g2[None], be2[None]

    buf0 = jnp.concatenate(
        [x_seq.reshape(P * N, IN),
         jnp.ones((CH, IN), jnp.float32),
         jnp.zeros((HOFF + TBL - P * N - CH, IN), jnp.float32)], axis=0)

    def step(k, carry):
        buf, s2a, pooled, deg = carry
        t = jnp.where(k == 0, 3, (k - 1) % 3)
        i = jnp.maximum(k - 1, 0) // 3
        off = jnp.where(t == 3, P * N,
                        jnp.where(t == 0, i * N, HOFF))
        tbl = lax.dynamic_slice(buf, (off, 0), (TBL, IN))
        idx_k = lax.dynamic_index_in_dim(IDX, t, 0, keepdims=False)
        dst_k = lax.dynamic_index_in_dim(DST, t, 0, keepdims=False)
        s = _agg_step(tbl, idx_k, dst_k)

        deg = lax.cond(
            t == 3,
            lambda dg: (s[0, :N, :1] + s[1, :N, :1]),
            lambda dg: dg, deg)
        buf = lax.cond(
            t == 0,
            lambda bf: lax.dynamic_update_slice(
                bf, _dense1(s, deg, tbl, Wl1, Wr1, b1, g1b,
                            be1b).reshape(TBL, IN), (HOFF, 0)),
            lambda bf: bf, buf)
        s2a = lax.cond(t == 1, lambda: s, lambda: s2a)
        pooled = lax.cond(
            t == 2,
            lambda pw: lax.dynamic_update_slice(
                pw,
                _dense2_pool(
                    s2a, s, deg,
                    lax.slice(buf, (HOFF, 0),
                              (HOFF + TBL, IN)).reshape(NC, NPAD, IN),
                    Wl2, Wr2, b2, g2b, be2b),
                (i, 0, 0)),
            lambda pw: pw, pooled)
        return buf, s2a, pooled, deg

    pooled0 = jnp.zeros((P, 1, HG), jnp.float32)
    s2a0 = jnp.zeros((NC, NPAD, FW), jnp.float32)
    deg0 = jnp.zeros((N, 1), jnp.float32)
    _, _, pooled, _ = lax.fori_loop(0, 3 * P + 1, step,
                                    (buf0, s2a0, pooled0, deg0))

    Hseq = pooled.reshape(B, T, HG)
    whp = jnp.concatenate([Wh, jnp.zeros((HT, 127), jnp.float32)], axis=1)
    bhp = jnp.concatenate([bh, jnp.zeros((127,), jnp.float32)])[None]
    y = _gru_head(Hseq, W_ih.T, W_hh.T, b_ih[None], b_hh[None], whp, bhp)
    return y[:, 0]


# speedup vs baseline: 1.4682x; 1.4682x over previous
import functools

import jax
import jax.numpy as jnp
from jax import lax
from jax.experimental import pallas as pl
from jax.experimental.pallas import tpu as pltpu
from jax.experimental.pallas import tpu_sc as plsc

N = 10000
E = 160000
IN = 128
HG = 256
HT = 256
B = 2
T = 4
P = B * T
RB = 400

NC, NS = 2, 16
NPAD = 10240
RPS = NPAD // NS
CH = 128
EP = 163840
EH = EP // 2
NCH = 40
KB = 4
PF = 2
ZR = 64
FW = 128
TBL = NC * NPAD
HOFF = P * N + (TBL - N)
NPH = NPAD // 2
NPH_PAD = NPH + CH
RPSH = NPH_PAD // NS
FPSH = NPH // NS

_MESH = plsc.VectorSubcoreMesh(core_axis_name="c", subcore_axis_name="s")
_SC_PARAMS = pltpu.CompilerParams(use_tc_tiling_on_sc=False)


def _agg_body(tab_hbm, idx_hbm, dst_hbm, out_hbm, gidx, didx0, didx1, rows,
              zbuf, acc, *sems):
    gsem, ssem = sems[:KB], sems[KB:]
    c = lax.axis_index("c")
    s = lax.axis_index("s")

    @pl.loop(0, ZR)
    def _zrow(r):
        @pl.loop(0, FW // 32)
        def _zcol(k):
            zbuf[r, pl.ds(k * 32, 32)] = jnp.zeros((32,), jnp.bfloat16)

    pltpu.sync_copy(idx_hbm.at[c, s], gidx)
    pltpu.sync_copy(dst_hbm.at[0, c, s], didx0)
    pltpu.sync_copy(dst_hbm.at[1, c, s], didx1)

    for q in range(2):
        for off in range(0, RPSH, ZR):
            sz = min(ZR, RPSH - off)
            pltpu.sync_copy(zbuf.at[pl.ds(0, sz)],
                            acc.at[pl.ds(s * RPSH + off, sz)])

        plsc.subcore_barrier()
        dq = didx0 if q == 0 else didx1

        for m in range(PF):
            pltpu.async_copy(tab_hbm.at[gidx.at[m]], rows.at[m], gsem[m])

        @pl.loop(0, NCH // KB)
        def _grp(jj):
            for b in range(KB):
                j = jj * KB + b
                pltpu.make_async_copy(tab_hbm.at[gidx.at[j]], rows.at[b],
                                      gsem[b]).wait()
                pltpu.async_copy(rows.at[b], acc.at[dq.at[j]], ssem[b],
                                 add=True)
                bm = (b + PF) % KB

                @pl.when(j + PF < NCH)
                def _next():
                    @pl.when(j + PF >= KB)
                    def _wait_prev():
                        pltpu.make_async_copy(rows.at[bm], acc.at[dq.at[0]],
                                              ssem[bm]).wait()

                    pltpu.async_copy(tab_hbm.at[gidx.at[j + PF]],
                                     rows.at[bm], gsem[bm])

        for b in range(KB):
            pltpu.make_async_copy(rows.at[b], acc.at[dq.at[0]],
                                  ssem[b]).wait()

        plsc.subcore_barrier()
        pltpu.sync_copy(acc.at[pl.ds(s * FPSH, FPSH)],
                        out_hbm.at[c, pl.ds(q * NPH + s * FPSH, FPSH)])
        plsc.subcore_barrier()


_agg_step = pl.kernel(
    _agg_body,
    out_type=jax.ShapeDtypeStruct((NC, NPAD, FW), jnp.bfloat16),
    mesh=_MESH,
    scratch_types=[
        pltpu.VMEM((NCH, CH), jnp.int32),
        pltpu.VMEM((NCH, CH), jnp.int32),
        pltpu.VMEM((NCH, CH), jnp.int32),
        pltpu.VMEM((KB, CH, FW), jnp.bfloat16),
        pltpu.VMEM((ZR, FW), jnp.bfloat16),
        pltpu.VMEM_SHARED((NPH_PAD, FW), jnp.bfloat16),
    ] + [pltpu.SemaphoreType.DMA] * (2 * KB),
    compiler_params=_SC_PARAMS,
)


def _dense1_body(s_ref, deg_ref, x_ref, wl_ref, wr_ref, b_ref, g_ref, be_ref,
                 o_ref):
    deg = jnp.maximum(deg_ref[...], 1.0)
    mean = (s_ref[0].astype(jnp.float32) + s_ref[1].astype(jnp.float32)) / deg
    z = (jnp.dot(mean, wl_ref[...], preferred_element_type=jnp.float32)
         + jnp.dot(x_ref[...].astype(jnp.float32), wr_ref[...],
                   preferred_element_type=jnp.float32)
         + b_ref[...])
    mu = jnp.mean(z, axis=-1, keepdims=True)
    var = jnp.mean((z - mu) ** 2, axis=-1, keepdims=True)
    z = (z - mu) * jax.lax.rsqrt(var + 1e-5) * g_ref[...] + be_ref[...]
    z = jnp.maximum(z, 0.0).astype(jnp.bfloat16)
    o_ref[0] = z[:, :128]
    o_ref[1] = z[:, 128:]


def _dense1(s, deg, x, wl, wr, b, g, be):
    return pl.pallas_call(
        _dense1_body,
        grid=(N // RB,),
        in_specs=[
            pl.BlockSpec((NC, RB, 128), lambda r: (0, r, 0)),
            pl.BlockSpec((RB, 1), lambda r: (r, 0)),
            pl.BlockSpec((RB, IN), lambda r: (r, 0)),
            pl.BlockSpec((IN, HG), lambda r: (0, 0)),
            pl.BlockSpec((IN, HG), lambda r: (0, 0)),
            pl.BlockSpec((1, HG), lambda r: (0, 0)),
            pl.BlockSpec((1, HG), lambda r: (0, 0)),
            pl.BlockSpec((1, HG), lambda r: (0, 0)),
        ],
        out_specs=pl.BlockSpec((NC, RB, 128), lambda r: (0, r, 0)),
        out_shape=jax.ShapeDtypeStruct((NC, NPAD, 128), jnp.bfloat16),
    )(s, deg, x, wl, wr, b, g, be)


def _dense2_body(sa_ref, sb_ref, deg_ref, h_ref, wl_ref, wr_ref, b_ref,
                 g_ref, be_ref, o_ref):
    r = pl.program_id(0)
    deg = jnp.maximum(deg_ref[...], 1.0)
    mean = (jnp.concatenate([sa_ref[0], sa_ref[1]], axis=1).astype(jnp.float32)
            + jnp.concatenate([sb_ref[0], sb_ref[1]],
                              axis=1).astype(jnp.float32)) / deg
    h = jnp.concatenate([h_ref[0], h_ref[1]], axis=1).astype(jnp.float32)
    z = (jnp.dot(mean, wl_ref[...], preferred_element_type=jnp.float32)
         + jnp.dot(h, wr_ref[...], preferred_element_type=jnp.float32)
         + b_ref[...])
    mu = jnp.mean(z, axis=-1, keepdims=True)
    var = jnp.mean((z - mu) ** 2, axis=-1, keepdims=True)
    z = (z - mu) * jax.lax.rsqrt(var + 1e-5) * g_ref[...] + be_ref[...]
    z = jnp.maximum(z, 0.0)
    part = jnp.sum(z, axis=0, keepdims=True)[None] * (1.0 / N)

    @pl.when(r == 0)
    def _():
        o_ref[...] = jnp.zeros_like(o_ref)

    o_ref[...] += part


def _dense2_pool(sa, sb, deg, h, wl, wr, b, g, be):
    return pl.pallas_call(
        _dense2_body,
        grid=(N // RB,),
        in_specs=[
            pl.BlockSpec((NC, RB, 128), lambda r: (0, r, 0)),
            pl.BlockSpec((NC, RB, 128), lambda r: (0, r, 0)),
            pl.BlockSpec((RB, 1), lambda r: (r, 0)),
            pl.BlockSpec((NC, RB, 128), lambda r: (0, r, 0)),
            pl.BlockSpec((HG, HG), lambda r: (0, 0)),
            pl.BlockSpec((HG, HG), lambda r: (0, 0)),
            pl.BlockSpec((1, HG), lambda r: (0, 0)),
            pl.BlockSpec((1, HG), lambda r: (0, 0)),
            pl.BlockSpec((1, HG), lambda r: (0, 0)),
        ],
        out_specs=pl.BlockSpec((1, 1, HG), lambda r: (0, 0, 0)),
        out_shape=jax.ShapeDtypeStruct((1, 1, HG), jnp.float32),
    )(sa, sb, deg, h, wl, wr, b, g, be)


def _gru_body(h_ref, wih_ref, whh_ref, bih_ref, bhh_ref, wh_ref, bh_ref,
              o_ref):
    h = jnp.zeros((B, HT), jnp.float32)
    for t in range(T):
        xt = h_ref[:, t, :]
        gi = jnp.dot(xt, wih_ref[...],
                     preferred_element_type=jnp.float32) + bih_ref[...]
        gh = jnp.dot(h, whh_ref[...],
                     preferred_element_type=jnp.float32) + bhh_ref[...]
        ir, iz, inn = gi[:, :HT], gi[:, HT:2 * HT], gi[:, 2 * HT:]
        hr, hz, hn = gh[:, :HT], gh[:, HT:2 * HT], gh[:, 2 * HT:]
        r = jax.nn.sigmoid(ir + hr)
        z = jax.nn.sigmoid(iz + hz)
        n = jnp.tanh(inn + r * hn)
        h = (1.0 - z) * n + z * h
    o_ref[...] = (jnp.dot(h, wh_ref[...], preferred_element_type=jnp.float32)
                  + bh_ref[...])


def _gru_head(Hseq, wihT, whhT, bih, bhh, whp, bhp):
    return pl.pallas_call(
        _gru_body,
        in_specs=[pl.BlockSpec(Hseq.shape, lambda: (0, 0, 0)),
                  pl.BlockSpec(wihT.shape, lambda: (0, 0)),
                  pl.BlockSpec(whhT.shape, lambda: (0, 0)),
                  pl.BlockSpec(bih.shape, lambda: (0, 0)),
                  pl.BlockSpec(bhh.shape, lambda: (0, 0)),
                  pl.BlockSpec(whp.shape, lambda: (0, 0)),
                  pl.BlockSpec(bhp.shape, lambda: (0, 0))],
        out_specs=pl.BlockSpec((B, 128), lambda: (0, 0)),
        out_shape=jax.ShapeDtypeStruct((B, 128), jnp.float32),
    )(Hseq, wihT, whhT, bih, bhh, whp, bhp)


def kernel(x_seq, edge_index, Wl1, bl1, Wr1, br1, g1, be1, Wl2, bl2, Wr2, br2,
           g2, be2, W_ih, W_hh, b_ih, b_hh, Wh, bh):
    src = edge_index[0].astype(jnp.int32)
    dst = edge_index[1].astype(jnp.int32)
    srcp = jnp.pad(src, (0, EP - E))
    dstp = jnp.pad(dst, (0, EP - E), constant_values=N)

    cc = jnp.arange(NC, dtype=jnp.int32)[:, None]
    ee = jnp.arange(EP, dtype=jnp.int32)
    idx_l1 = srcp.reshape(NC, NS, NCH, CH)
    idx_2a = (cc * NPAD + srcp[None, :EH]).reshape(NC, NS, NCH, CH)
    idx_2b = (cc * NPAD + srcp[None, EH:]).reshape(NC, NS, NCH, CH)
    idx_dg = (ee % CH).reshape(NC, NS, NCH, CH)
    IDX = jnp.stack([idx_l1, idx_2a, idx_2b, idx_dg])

    qq = jnp.arange(2, dtype=jnp.int32)[:, None]
    owned = (dstp[None, :] >= qq * NPH) & (dstp[None, :] < (qq + 1) * NPH)
    dstq = jnp.where(owned, dstp[None, :] - qq * NPH,
                     NPH + (ee[None, :] % CH))
    dst_l1 = dstq.reshape(2, NC, NS, NCH, CH)
    dst_2a = jnp.broadcast_to(dstq[:, :EH].reshape(2, 1, NS, NCH, CH),
                              (2, NC, NS, NCH, CH))
    dst_2b = jnp.broadcast_to(dstq[:, EH:].reshape(2, 1, NS, NCH, CH),
                              (2, NC, NS, NCH, CH))
    DST = jnp.stack([dst_l1, dst_2a, dst_2b, dst_l1])

    b1 = (bl1 + br1)[None]
    b2 = (bl2 + br2)[None]
    g1b, be1b, g2b, be2b = g1[None], be1[None], g2[None], be2[None]

    buf0 = jnp.concatenate(
        [x_seq.reshape(P * N, IN).astype(jnp.bfloat16),
         jnp.ones((CH, IN), jnp.bfloat16),
         jnp.zeros((HOFF + TBL - P * N - CH, IN), jnp.bfloat16)], axis=0)

    def step(k, carry):
        buf, s2a, pooled, deg = carry
        t = jnp.where(k == 0, 3, (k - 1) % 3)
        i = jnp.maximum(k - 1, 0) // 3
        off = jnp.where(t == 3, P * N,
                        jnp.where(t == 0, i * N, HOFF))
        tbl = lax.dynamic_slice(buf, (off, 0), (TBL, IN))
        idx_k = lax.dynamic_index_in_dim(IDX, t, 0, keepdims=False)
        dst_k = lax.dynamic_index_in_dim(DST, t, 0, keepdims=False)
        s = _agg_step(tbl, idx_k, dst_k)

        deg = lax.cond(
            t == 3,
            lambda dg: (s[0, :N, :1].astype(jnp.float32)
                        + s[1, :N, :1].astype(jnp.float32)),
            lambda dg: dg, deg)
        buf = lax.cond(
            t == 0,
            lambda bf: lax.dynamic_update_slice(
                bf, _dense1(s, deg, tbl, Wl1, Wr1, b1, g1b,
                            be1b).reshape(TBL, IN), (HOFF, 0)),
            lambda bf: bf, buf)
        s2a = lax.cond(t == 1, lambda: s, lambda: s2a)
        pooled = lax.cond(
            t == 2,
            lambda pw: lax.dynamic_update_slice(
                pw,
                _dense2_pool(
                    s2a, s, deg,
                    lax.slice(buf, (HOFF, 0),
                              (HOFF + TBL, IN)).reshape(NC, NPAD, IN),
                    Wl2, Wr2, b2, g2b, be2b),
                (i, 0, 0)),
            lambda pw: pw, pooled)
        return buf, s2a, pooled, deg

    pooled0 = jnp.zeros((P, 1, HG), jnp.float32)
    s2a0 = jnp.zeros((NC, NPAD, FW), jnp.bfloat16)
    deg0 = jnp.zeros((N, 1), jnp.float32)
    _, _, pooled, _ = lax.fori_loop(0, 3 * P + 1, step,
                                    (buf0, s2a0, pooled0, deg0))

    Hseq = pooled.reshape(B, T, HG)
    whp = jnp.concatenate([Wh, jnp.zeros((HT, 127), jnp.float32)], axis=1)
    bhp = jnp.concatenate([bh, jnp.zeros((127,), jnp.float32)])[None]
    y = _gru_head(Hseq, W_ih.T, W_hh.T, b_ih[None], b_hh[None], whp, bhp)
    return y[:, 0]
